# baseline (speedup 1.0000x reference)
import functools

import jax
import jax.numpy as jnp
from jax import lax
from jax.experimental import pallas as pl
from jax.experimental.pallas import tpu as pltpu
from jax.experimental.pallas import tpu_sc as plsc

_IT = False

N = 10000
E = 320000
H = 128
NOISE = 16
B = 16

NC = 2
NS = 16
NW = NC * NS
CH = 80
NCH = E // CH
E2 = 327680
BE = 8192
NEB = E2 // BE
NR = 1000
NPAD = 10240


def _pp_body(et_W, P_W, b1in, time_emb, t_W, t_b, batch_pf, W1, b1, te, starts):
    W1[...] = jnp.dot(et_W[...], P_W[...], preferred_element_type=jnp.float32)
    b1[...] = jnp.dot(b1in[...], P_W[...], preferred_element_type=jnp.float32)
    te[...] = (
        jnp.dot(jax.nn.relu(time_emb[...]), t_W[...],
                preferred_element_type=jnp.float32) + t_b[...])
    iota = lax.broadcasted_iota(jnp.int32, (NPAD, H), 1).astype(jnp.float32)
    ohlt = (batch_pf[...] < iota).astype(jnp.float32)
    starts[...] = jnp.dot(jnp.ones((1, NPAD), jnp.float32), ohlt,
                          preferred_element_type=jnp.float32)


def _bseg_of(src_blk, starts_ref):
    sf = src_blk.astype(jnp.float32)
    acc = jnp.zeros(src_blk.shape, jnp.int32)
    for b in range(1, B):
        acc += (sf >= starts_ref[0, b]).astype(jnp.int32)
    return acc


def _node_body(x, qW, rW, uW, vW, qb, rb, ub, vb, qh, rh, uh, vh):
    xx = x[...]
    qh[...] = jnp.dot(xx, qW[...], preferred_element_type=jnp.float32) + qb[...]
    rh[...] = jnp.dot(xx, rW[...], preferred_element_type=jnp.float32) + rb[...]
    uh[...] = jnp.dot(xx, uW[...], preferred_element_type=jnp.float32) + ub[...]
    vh[...] = jnp.dot(xx, vW[...], preferred_element_type=jnp.float32) + vb[...]


_DNT = (((0,), (0,)), ((), ()))


def _a_body(nmeT, W1, b1, a):
    a[...] = (lax.dot_general(nmeT[...], W1[...], _DNT,
                              preferred_element_type=jnp.float32) + b1[...])


def _sc_body(a_hbm, qh_hbm, rh_hbm, vh_hbm, src_hbm, dst_hbm,
             zeros_hbm,
             ehat_hbm, aggr_hbm,
             srcb, dstb, abuf, vbuf,
             aggr_sp, sem_i, sem_g):
    c = lax.axis_index("c")
    s = lax.axis_index("s")
    wid = s * NC + c

    @pl.when(s == 0)
    def _():
        pltpu.sync_copy(zeros_hbm, aggr_sp)

    plsc.subcore_barrier()

    def base_of(g):
        return (wid + NW * g) * CH

    def cond(g):
        return wid + NW * g < NCH

    def fire_ia(g, b):
        base = base_of(g)
        pltpu.async_copy(src_hbm.at[pl.ds(base, CH)], srcb.at[b], sem_i)
        pltpu.async_copy(dst_hbm.at[pl.ds(base, CH)], dstb.at[b], sem_i)
        pltpu.async_copy(a_hbm.at[pl.ds(base, CH)], abuf.at[b], sem_i)

    def wait_ia(b):
        pltpu.make_async_copy(src_hbm.at[pl.ds(0, CH)], srcb.at[b],
                              sem_i).wait()
        pltpu.make_async_copy(dst_hbm.at[pl.ds(0, CH)], dstb.at[b],
                              sem_i).wait()
        pltpu.make_async_copy(a_hbm.at[pl.ds(0, CH)], abuf.at[b],
                              sem_i).wait()

    def fire_gathers(b):
        pltpu.async_copy(qh_hbm.at[srcb.at[b]], abuf.at[b], sem_g, add=True)
        pltpu.async_copy(rh_hbm.at[dstb.at[b]], abuf.at[b], sem_g, add=True)
        pltpu.async_copy(vh_hbm.at[dstb.at[b]], vbuf.at[b], sem_g)

    def wait_gathers(b):
        for _ in range(3):
            pltpu.make_async_copy(qh_hbm.at[pl.ds(0, CH)], vbuf.at[b],
                                  sem_g).wait()

    fire_ia(0, 0)
    wait_ia(0)
    fire_gathers(0)
    fire_ia(1, 1)

    def pair_body(i, _):
        for b in range(2):
            g = 2 * i + b

            @pl.when(cond(g))
            def _():
                wait_gathers(b)

                def edge_body(e, _):
                    for j in range(H // 16):
                        sl = pl.ds(16 * j, 16)
                        vbuf[b, e, sl] = vbuf[b, e, sl] / (
                            1.0 + jnp.exp(-abuf[b, e, sl]))
                    return 0

                lax.fori_loop(0, CH, edge_body, 0, unroll=4)
                base = base_of(g)
                pltpu.sync_copy(abuf.at[b], ehat_hbm.at[pl.ds(base, CH)])
                pltpu.sync_copy(vbuf.at[b], aggr_sp.at[srcb.at[b]], add=True)

            @pl.when(cond(g + 1))
            def _():
                wait_ia(1 - b)
                fire_gathers(1 - b)

            @pl.when(cond(g + 2))
            def _():
                fire_ia(g + 2, b)
        return 0

    lax.fori_loop(0, (NCH // NW + 2) // 2, pair_body, 0)

    plsc.subcore_barrier()

    @pl.when(s == 0)
    def _():
        pltpu.sync_copy(aggr_sp, aggr_hbm.at[c])


def _stats_body(ehat, src, starts, ssum, ssq, cnt):
    i = pl.program_id(0)

    @pl.when(i == 0)
    def _():
        ssum[...] = jnp.zeros((B, H), jnp.float32)
        ssq[...] = jnp.zeros((B, H), jnp.float32)
        cnt[...] = jnp.zeros((B, H), jnp.float32)

    sv = src[...]
    sv2 = sv[:, None]
    eb = jnp.where(sv2 >= 0, ehat[...], 0.0)
    seg = _bseg_of(sv, starts)
    iota = lax.broadcasted_iota(jnp.int32, (BE, B), 1)
    oh = ((seg[:, None] == iota) & (sv2 >= 0)).astype(jnp.float32)
    dn = (((0,), (0,)), ((), ()))
    ssum[...] += lax.dot_general(oh, eb, dn, preferred_element_type=jnp.float32)
    ssq[...] += lax.dot_general(oh, eb * eb, dn,
                                preferred_element_type=jnp.float32)
    cnt[...] += jnp.broadcast_to(jnp.sum(oh, axis=0)[:, None], (B, H))


def _h_body(feat, uh, p0, p1, batch, gw, gb, gms, hout):
    x = uh[...] + p0[...] + p1[...]
    seg = batch[...][:N]
    iota = lax.broadcasted_iota(jnp.int32, (N, B), 1)
    oh = (seg[:, None] == iota).astype(jnp.float32)
    dn = (((0,), (0,)), ((), ()))
    cntc = jnp.maximum(jnp.sum(oh, axis=0), 1.0)[:, None]
    mean = lax.dot_general(oh, x, dn, preferred_element_type=jnp.float32) / cntc
    out = x - jnp.dot(oh, mean, preferred_element_type=jnp.float32) * gms[...]
    var = lax.dot_general(oh, out * out, dn,
                          preferred_element_type=jnp.float32) / cntc
    rstd = 1.0 / jnp.sqrt(var + 1e-5)
    gn = out * jnp.dot(oh, rstd, preferred_element_type=jnp.float32) * gw[...] \
        + gb[...]
    hout[...] = feat[...] + jax.nn.relu(gn)


def _f_body(ehat, src, starts, nmeT, ssum, ssq, cnt, te, gw, gb, gms, lnw, lnb,
            oW, ob, etW, etb, eout):
    cntc = jnp.maximum(cnt[...], 1.0)
    m = ssum[...] / cntc
    ms = gms[...]
    var = ssq[...] / cntc - 2.0 * ms * m * m + ms * ms * m * m
    rstd = 1.0 / jnp.sqrt(var + 1e-5)
    ascale = rstd * gw[...]
    coff = gb[...] - m * ms * ascale

    seg = _bseg_of(src[...], starts)
    iota = lax.broadcasted_iota(jnp.int32, (BE, B), 1)
    oh = (seg[:, None] == iota).astype(jnp.float32)
    x = ehat[...] * jnp.dot(oh, ascale, preferred_element_type=jnp.float32) \
        + jnp.dot(oh, coff, preferred_element_type=jnp.float32)
    x = jax.nn.relu(x)
    x = x + jnp.dot(oh, te[...], preferred_element_type=jnp.float32)

    mu = jnp.mean(x, axis=-1, keepdims=True)
    vv = jnp.mean((x - mu) ** 2, axis=-1, keepdims=True)
    ln = (x - mu) / jnp.sqrt(vv + 1e-5) * lnw[...] + lnb[...]
    sl = ln * jax.nn.sigmoid(ln)
    y = jnp.dot(sl.astype(jnp.bfloat16), oW[...].astype(jnp.bfloat16),
                preferred_element_type=jnp.float32) + ob[...]
    noise = lax.dot_general(nmeT[...], etW[...], _DNT,
                            preferred_element_type=jnp.float32) + etb[...]
    eout[...] = noise + y


def _sc_stage(a, qh, rh, vh, src, dst):
    f32 = jnp.float32
    mesh = plsc.VectorSubcoreMesh(core_axis_name="c", subcore_axis_name="s",
                                  num_cores=NC, num_subcores=NS)
    sc = functools.partial(
        pl.kernel, mesh=mesh,
        out_type=[jax.ShapeDtypeStruct((E2, H), f32),
                  jax.ShapeDtypeStruct((NC, N, H), f32)],
        scratch_types=[
            pltpu.VMEM((2, CH), jnp.int32),
            pltpu.VMEM((2, CH), jnp.int32),
            pltpu.VMEM((2, CH, H), f32),
            pltpu.VMEM((2, CH, H), f32),
            pltpu.VMEM_SHARED((N, H), f32),
            pltpu.SemaphoreType.DMA,
            pltpu.SemaphoreType.DMA,
        ],
    )(_sc_body)
    zeros = jnp.zeros((N, H), f32)
    return sc(a, qh, rh, vh, src, dst, zeros)


def _full(shape):
    return pl.BlockSpec(shape, lambda *_: tuple(0 for _ in shape))


def kernel(features, edge_mapping_idx, noise_mapping_emb, time_emb, batch,
           params):
    p = params
    f32 = jnp.float32
    src = edge_mapping_idx[0]
    dst = edge_mapping_idx[1]
    r1 = lambda a: a.reshape(1, -1)

    batch_pf = jnp.pad(batch, (0, NPAD - N),
                       constant_values=B + 1).astype(f32).reshape(NPAD, 1)
    W1, b1, te, starts = pl.pallas_call(
        _pp_body,
        grid=(1,),
        in_specs=[_full((NOISE, H)), _full((H, H)), _full((1, H)),
                  _full((B, H)), _full((H, H)), _full((1, H)),
                  _full((NPAD, 1))],
        out_specs=[_full((NOISE, H)), _full((1, H)), _full((B, H)),
                   _full((1, H))],
        out_shape=[jax.ShapeDtypeStruct((NOISE, H), f32),
                   jax.ShapeDtypeStruct((1, H), f32),
                   jax.ShapeDtypeStruct((B, H), f32),
                   jax.ShapeDtypeStruct((1, H), f32)],
        interpret=_IT,
    )(p['et_W'], p['P_W'], r1(p['et_b']), time_emb, p['t_W'], r1(p['t_b']),
      batch_pf)
    b1 = b1 + r1(p['P_b'])

    row = pl.BlockSpec((NR, H), lambda i: (i, 0))
    qh, rh, uh, vh = pl.pallas_call(
        _node_body,
        grid=(N // NR,),
        in_specs=[row] + [_full((H, H))] * 4 + [_full((1, H))] * 4,
        out_specs=[row] * 4,
        out_shape=[jax.ShapeDtypeStruct((N, H), f32)] * 4,
        interpret=_IT,
    )(features, p['Q_W'], p['R_W'], p['U_W'], p['V_W'],
      r1(p['Q_b']), r1(p['R_b']), r1(p['U_b']), r1(p['V_b']))

    src_p = jnp.pad(src, (0, E2 - E), constant_values=-1)
    nmeT = noise_mapping_emb.T
    ncol = pl.BlockSpec((NOISE, BE), lambda i: (0, i))
    a = pl.pallas_call(
        _a_body,
        grid=(NEB,),
        in_specs=[ncol, _full((NOISE, H)), _full((1, H))],
        out_specs=pl.BlockSpec((BE, H), lambda i: (i, 0)),
        out_shape=jax.ShapeDtypeStruct((E2, H), f32),
        interpret=_IT,
    )(nmeT, W1, b1)

    ehat, aggr2 = _sc_stage(a, qh, rh, vh, src, dst)

    erow = pl.BlockSpec((BE, H), lambda i: (i, 0))
    irow = pl.BlockSpec((BE,), lambda i: (i,))
    acc = pl.BlockSpec((B, H), lambda i: (0, 0))
    ssum, ssq, cnt = pl.pallas_call(
        _stats_body,
        grid=(NEB,),
        in_specs=[erow, irow, _full((1, H))],
        out_specs=[acc] * 3,
        out_shape=[jax.ShapeDtypeStruct((B, H), f32)] * 3,
        interpret=_IT,
    )(ehat, src_p, starts)

    batch_p = jnp.pad(batch, (0, NPAD - N), constant_values=B + 1)
    hout = pl.pallas_call(
        _h_body,
        grid=(1,),
        in_specs=[_full((N, H))] * 4 + [_full((NPAD,))] + [_full((1, H))] * 3,
        out_specs=_full((N, H)),
        out_shape=jax.ShapeDtypeStruct((N, H), f32),
        interpret=_IT,
    )(features, uh, aggr2[0], aggr2[1], batch_p,
      r1(p['gn_h_w']), r1(p['gn_h_b']), r1(p['gn_h_ms']))

    eout = pl.pallas_call(
        _f_body,
        grid=(NEB,),
        in_specs=[erow, irow, _full((1, H)), ncol,
                  _full((B, H)), _full((B, H)), _full((B, H)), _full((B, H)),
                  _full((1, H)), _full((1, H)), _full((1, H)),
                  _full((1, H)), _full((1, H)),
                  _full((H, H)), _full((1, H)),
                  _full((NOISE, H)), _full((1, H))],
        out_specs=erow,
        out_shape=jax.ShapeDtypeStruct((E, H), f32),
        interpret=_IT,
    )(ehat, src_p, starts, nmeT, ssum, ssq, cnt, te,
      r1(p['gn_e_w']), r1(p['gn_e_b']), r1(p['gn_e_ms']),
      r1(p['ln_w']), r1(p['ln_b']), p['o_W'], r1(p['o_b']),
      p['et_W'], r1(p['et_b']))

    return (hout, eout)


# speedup vs baseline: 2.2859x; 2.2859x over previous
import functools

import jax
import jax.numpy as jnp
from jax import lax
from jax.experimental import pallas as pl
from jax.experimental.pallas import tpu as pltpu
from jax.experimental.pallas import tpu_sc as plsc

_IT = False

N = 10000
E = 320000
H = 128
NOISE = 16
B = 16

NC = 2
NS = 16
NW = NC * NS
CH = 80
NCH = E // CH
E2 = 327680
BE = 8192
NEB = E2 // BE
NR = 1000
NPAD = 10240


def _pp_body(et_W, P_W, b1in, time_emb, t_W, t_b, batch_pf, W1, b1, te, starts):
    W1[...] = jnp.dot(et_W[...], P_W[...], preferred_element_type=jnp.float32)
    b1[...] = jnp.dot(b1in[...], P_W[...], preferred_element_type=jnp.float32)
    te[...] = (
        jnp.dot(jax.nn.relu(time_emb[...]), t_W[...],
                preferred_element_type=jnp.float32) + t_b[...])
    iota = lax.broadcasted_iota(jnp.int32, (NPAD, H), 1).astype(jnp.float32)
    ohlt = (batch_pf[...] < iota).astype(jnp.float32)
    starts[...] = jnp.dot(jnp.ones((1, NPAD), jnp.float32), ohlt,
                          preferred_element_type=jnp.float32)


def _bseg_of(src_blk, starts_ref):
    sf = src_blk.astype(jnp.float32)
    acc = jnp.zeros(src_blk.shape, jnp.int32)
    for b in range(1, B):
        acc += (sf >= starts_ref[0, b]).astype(jnp.int32)
    return acc


def _node_body(x, qW, rW, uW, vW, qb, rb, ub, vb, qh, rh, uh, vh):
    xx = x[...]
    qh[...] = jnp.dot(xx, qW[...], preferred_element_type=jnp.float32) + qb[...]
    rh[...] = jnp.dot(xx, rW[...], preferred_element_type=jnp.float32) + rb[...]
    uh[...] = jnp.dot(xx, uW[...], preferred_element_type=jnp.float32) + ub[...]
    vh[...] = jnp.dot(xx, vW[...], preferred_element_type=jnp.float32) + vb[...]


_DNT = (((0,), (0,)), ((), ()))


def _a_body(nmeT, W1, b1, a):
    a[...] = (lax.dot_general(nmeT[...], W1[...], _DNT,
                              preferred_element_type=jnp.float32) + b1[...])


def _sc_body(a_hbm, qh_hbm, rh_hbm, vh_hbm, src_hbm, dst_hbm,
             zeros_hbm,
             ehat_hbm, aggr_hbm,
             srcb, dstb, abuf, vbuf,
             aggr_sp, sem_i, sem_g):
    c = lax.axis_index("c")
    s = lax.axis_index("s")
    wid = s * NC + c

    @pl.when(s == 0)
    def _():
        pltpu.sync_copy(zeros_hbm, aggr_sp)

    plsc.subcore_barrier()

    def base_of(g):
        return (wid + NW * g) * CH

    def cond(g):
        return wid + NW * g < NCH

    def fire_ia(g, b):
        base = base_of(g)
        pltpu.async_copy(src_hbm.at[pl.ds(base, CH)], srcb.at[b], sem_i)
        pltpu.async_copy(dst_hbm.at[pl.ds(base, CH)], dstb.at[b], sem_i)
        pltpu.async_copy(a_hbm.at[pl.ds(base, CH)], abuf.at[b], sem_i)

    def wait_ia(b):
        pltpu.make_async_copy(src_hbm.at[pl.ds(0, CH)], srcb.at[b],
                              sem_i).wait()
        pltpu.make_async_copy(dst_hbm.at[pl.ds(0, CH)], dstb.at[b],
                              sem_i).wait()
        pltpu.make_async_copy(a_hbm.at[pl.ds(0, CH)], abuf.at[b],
                              sem_i).wait()

    def fire_gathers(b):
        pltpu.async_copy(qh_hbm.at[srcb.at[b]], abuf.at[b], sem_g, add=True)
        pltpu.async_copy(rh_hbm.at[dstb.at[b]], abuf.at[b], sem_g, add=True)
        pltpu.async_copy(vh_hbm.at[dstb.at[b]], vbuf.at[b], sem_g)

    def wait_gathers(b):
        for _ in range(3):
            pltpu.make_async_copy(qh_hbm.at[pl.ds(0, CH)], vbuf.at[b],
                                  sem_g).wait()

    fire_ia(0, 0)
    wait_ia(0)
    fire_gathers(0)
    fire_ia(1, 1)

    def pair_body(i, _):
        for b in range(2):
            g = 2 * i + b

            @pl.when(cond(g))
            def _():
                wait_gathers(b)

                def edge_body(e, _):
                    for j in range(H // 16):
                        sl = pl.ds(16 * j, 16)
                        vbuf[b, e, sl] = vbuf[b, e, sl] / (
                            1.0 + jnp.exp(-abuf[b, e, sl]))
                    return 0

                lax.fori_loop(0, CH, edge_body, 0)
                base = base_of(g)
                pltpu.sync_copy(abuf.at[b], ehat_hbm.at[pl.ds(base, CH)])
                pltpu.sync_copy(vbuf.at[b], aggr_sp.at[srcb.at[b]], add=True)

            @pl.when(cond(g + 1))
            def _():
                wait_ia(1 - b)
                fire_gathers(1 - b)

            @pl.when(cond(g + 2))
            def _():
                fire_ia(g + 2, b)
        return 0

    lax.fori_loop(0, (NCH // NW + 2) // 2, pair_body, 0)

    plsc.subcore_barrier()

    @pl.when(s == 0)
    def _():
        pltpu.sync_copy(aggr_sp, aggr_hbm.at[c])


def _stats_body(ehat, src, starts, ssum, ssq, cnt):
    i = pl.program_id(0)

    @pl.when(i == 0)
    def _():
        ssum[...] = jnp.zeros((B, H), jnp.float32)
        ssq[...] = jnp.zeros((B, H), jnp.float32)
        cnt[...] = jnp.zeros((B, H), jnp.float32)

    sv = src[...]
    sv2 = sv[:, None]
    eb = jnp.where(sv2 >= 0, ehat[...], 0.0)
    seg = _bseg_of(sv, starts)
    iota = lax.broadcasted_iota(jnp.int32, (BE, B), 1)
    oh = ((seg[:, None] == iota) & (sv2 >= 0)).astype(jnp.float32)
    dn = (((0,), (0,)), ((), ()))
    ssum[...] += lax.dot_general(oh, eb, dn, preferred_element_type=jnp.float32)
    ssq[...] += lax.dot_general(oh, eb * eb, dn,
                                preferred_element_type=jnp.float32)
    cnt[...] += jnp.broadcast_to(jnp.sum(oh, axis=0)[:, None], (B, H))


def _h_body(feat, uh, p0, p1, batch, gw, gb, gms, hout):
    x = uh[...] + p0[...] + p1[...]
    seg = batch[...][:N]
    iota = lax.broadcasted_iota(jnp.int32, (N, B), 1)
    oh = (seg[:, None] == iota).astype(jnp.float32)
    dn = (((0,), (0,)), ((), ()))
    cntc = jnp.maximum(jnp.sum(oh, axis=0), 1.0)[:, None]
    mean = lax.dot_general(oh, x, dn, preferred_element_type=jnp.float32) / cntc
    out = x - jnp.dot(oh, mean, preferred_element_type=jnp.float32) * gms[...]
    var = lax.dot_general(oh, out * out, dn,
                          preferred_element_type=jnp.float32) / cntc
    rstd = 1.0 / jnp.sqrt(var + 1e-5)
    gn = out * jnp.dot(oh, rstd, preferred_element_type=jnp.float32) * gw[...] \
        + gb[...]
    hout[...] = feat[...] + jax.nn.relu(gn)


def _f_body(ehat, src, starts, nmeT, ssum, ssq, cnt, te, gw, gb, gms, lnw, lnb,
            oW, ob, etW, etb, eout):
    cntc = jnp.maximum(cnt[...], 1.0)
    m = ssum[...] / cntc
    ms = gms[...]
    var = ssq[...] / cntc - 2.0 * ms * m * m + ms * ms * m * m
    rstd = 1.0 / jnp.sqrt(var + 1e-5)
    ascale = rstd * gw[...]
    coff = gb[...] - m * ms * ascale

    seg = _bseg_of(src[...], starts)
    iota = lax.broadcasted_iota(jnp.int32, (BE, B), 1)
    oh = (seg[:, None] == iota).astype(jnp.float32)
    x = ehat[...] * jnp.dot(oh, ascale, preferred_element_type=jnp.float32) \
        + jnp.dot(oh, coff, preferred_element_type=jnp.float32)
    x = jax.nn.relu(x)
    x = x + jnp.dot(oh, te[...], preferred_element_type=jnp.float32)

    mu = jnp.mean(x, axis=-1, keepdims=True)
    vv = jnp.mean((x - mu) ** 2, axis=-1, keepdims=True)
    ln = (x - mu) / jnp.sqrt(vv + 1e-5) * lnw[...] + lnb[...]
    sl = ln * jax.nn.sigmoid(ln)
    y = jnp.dot(sl.astype(jnp.bfloat16), oW[...].astype(jnp.bfloat16),
                preferred_element_type=jnp.float32) + ob[...]
    noise = lax.dot_general(nmeT[...], etW[...], _DNT,
                            preferred_element_type=jnp.float32) + etb[...]
    eout[...] = noise + y


def _sc_stage(a, qh, rh, vh, src, dst):
    f32 = jnp.float32
    mesh = plsc.VectorSubcoreMesh(core_axis_name="c", subcore_axis_name="s",
                                  num_cores=NC, num_subcores=NS)
    sc = functools.partial(
        pl.kernel, mesh=mesh,
        out_type=[jax.ShapeDtypeStruct((E2, H), f32),
                  jax.ShapeDtypeStruct((NC, N, H), f32)],
        scratch_types=[
            pltpu.VMEM((2, CH), jnp.int32),
            pltpu.VMEM((2, CH), jnp.int32),
            pltpu.VMEM((2, CH, H), f32),
            pltpu.VMEM((2, CH, H), f32),
            pltpu.VMEM_SHARED((N, H), f32),
            pltpu.SemaphoreType.DMA,
            pltpu.SemaphoreType.DMA,
        ],
    )(_sc_body)
    zeros = jnp.zeros((N, H), f32)
    return sc(a, qh, rh, vh, src, dst, zeros)


def _full(shape):
    return pl.BlockSpec(shape, lambda *_: tuple(0 for _ in shape))


def kernel(features, edge_mapping_idx, noise_mapping_emb, time_emb, batch,
           params):
    p = params
    f32 = jnp.float32
    src = edge_mapping_idx[0]
    dst = edge_mapping_idx[1]
    r1 = lambda a: a.reshape(1, -1)

    batch_pf = jnp.pad(batch, (0, NPAD - N),
                       constant_values=B + 1).astype(f32).reshape(NPAD, 1)
    W1, b1, te, starts = pl.pallas_call(
        _pp_body,
        grid=(1,),
        in_specs=[_full((NOISE, H)), _full((H, H)), _full((1, H)),
                  _full((B, H)), _full((H, H)), _full((1, H)),
                  _full((NPAD, 1))],
        out_specs=[_full((NOISE, H)), _full((1, H)), _full((B, H)),
                   _full((1, H))],
        out_shape=[jax.ShapeDtypeStruct((NOISE, H), f32),
                   jax.ShapeDtypeStruct((1, H), f32),
                   jax.ShapeDtypeStruct((B, H), f32),
                   jax.ShapeDtypeStruct((1, H), f32)],
        interpret=_IT,
    )(p['et_W'], p['P_W'], r1(p['et_b']), time_emb, p['t_W'], r1(p['t_b']),
      batch_pf)
    b1 = b1 + r1(p['P_b'])

    row = pl.BlockSpec((NR, H), lambda i: (i, 0))
    qh, rh, uh, vh = pl.pallas_call(
        _node_body,
        grid=(N // NR,),
        in_specs=[row] + [_full((H, H))] * 4 + [_full((1, H))] * 4,
        out_specs=[row] * 4,
        out_shape=[jax.ShapeDtypeStruct((N, H), f32)] * 4,
        interpret=_IT,
    )(features, p['Q_W'], p['R_W'], p['U_W'], p['V_W'],
      r1(p['Q_b']), r1(p['R_b']), r1(p['U_b']), r1(p['V_b']))

    src_p = jnp.pad(src, (0, E2 - E), constant_values=-1)
    nmeT = noise_mapping_emb.T
    ncol = pl.BlockSpec((NOISE, BE), lambda i: (0, i))
    a = pl.pallas_call(
        _a_body,
        grid=(NEB,),
        in_specs=[ncol, _full((NOISE, H)), _full((1, H))],
        out_specs=pl.BlockSpec((BE, H), lambda i: (i, 0)),
        out_shape=jax.ShapeDtypeStruct((E2, H), f32),
        interpret=_IT,
    )(nmeT, W1, b1)

    ehat, aggr2 = _sc_stage(a, qh, rh, vh, src, dst)

    erow = pl.BlockSpec((BE, H), lambda i: (i, 0))
    irow = pl.BlockSpec((BE,), lambda i: (i,))
    acc = pl.BlockSpec((B, H), lambda i: (0, 0))
    ssum, ssq, cnt = pl.pallas_call(
        _stats_body,
        grid=(NEB,),
        in_specs=[erow, irow, _full((1, H))],
        out_specs=[acc] * 3,
        out_shape=[jax.ShapeDtypeStruct((B, H), f32)] * 3,
        interpret=_IT,
    )(ehat, src_p, starts)

    batch_p = jnp.pad(batch, (0, NPAD - N), constant_values=B + 1)
    hout = pl.pallas_call(
        _h_body,
        grid=(1,),
        in_specs=[_full((N, H))] * 4 + [_full((NPAD,))] + [_full((1, H))] * 3,
        out_specs=_full((N, H)),
        out_shape=jax.ShapeDtypeStruct((N, H), f32),
        interpret=_IT,
    )(features, uh, aggr2[0], aggr2[1], batch_p,
      r1(p['gn_h_w']), r1(p['gn_h_b']), r1(p['gn_h_ms']))

    eout = pl.pallas_call(
        _f_body,
        grid=(NEB,),
        in_specs=[erow, irow, _full((1, H)), ncol,
                  _full((B, H)), _full((B, H)), _full((B, H)), _full((B, H)),
                  _full((1, H)), _full((1, H)), _full((1, H)),
                  _full((1, H)), _full((1, H)),
                  _full((H, H)), _full((1, H)),
                  _full((NOISE, H)), _full((1, H))],
        out_specs=erow,
        out_shape=jax.ShapeDtypeStruct((E, H), f32),
        interpret=_IT,
    )(ehat, src_p, starts, nmeT, ssum, ssq, cnt, te,
      r1(p['gn_e_w']), r1(p['gn_e_b']), r1(p['gn_e_ms']),
      r1(p['ln_w']), r1(p['ln_b']), p['o_W'], r1(p['o_b']),
      p['et_W'], r1(p['et_b']))

    return (hout, eout)


# speedup vs baseline: 2.6766x; 1.1709x over previous
import functools

import jax
import jax.numpy as jnp
from jax import lax
from jax.experimental import pallas as pl
from jax.experimental.pallas import tpu as pltpu
from jax.experimental.pallas import tpu_sc as plsc


N = 10000
E = 320000
H = 128
NOISE = 16
B = 16

NC = 2
NS = 16
NW = NC * NS
CH = 80
NCH = E // CH
E2 = 327680
BE = 8192
NEB = E2 // BE
NR = 1000
NPAD = 10240


def _pp_body(et_W, P_W, b1in, time_emb, t_W, t_b, batch_pf, W1, b1, te, starts):
    W1[...] = jnp.dot(et_W[...], P_W[...], preferred_element_type=jnp.float32)
    b1[...] = jnp.dot(b1in[...], P_W[...], preferred_element_type=jnp.float32)
    te[...] = (
        jnp.dot(jax.nn.relu(time_emb[...]), t_W[...],
                preferred_element_type=jnp.float32) + t_b[...])
    iota = lax.broadcasted_iota(jnp.int32, (NPAD, H), 1).astype(jnp.float32)
    ohlt = (batch_pf[...] < iota).astype(jnp.float32)
    starts[...] = jnp.dot(jnp.ones((1, NPAD), jnp.float32), ohlt,
                          preferred_element_type=jnp.float32)


def _bseg_of(src_blk, starts_ref):
    sf = src_blk.astype(jnp.float32)
    acc = jnp.zeros(src_blk.shape, jnp.int32)
    for b in range(1, B):
        acc += (sf >= starts_ref[0, b]).astype(jnp.int32)
    return acc


def _node_body(x, qW, rW, uW, vW, qb, rb, ub, vb, qh, rh, uh, vh):
    xx = x[...]
    qh[...] = jnp.dot(xx, qW[...], preferred_element_type=jnp.float32) + qb[...]
    rh[...] = jnp.dot(xx, rW[...], preferred_element_type=jnp.float32) + rb[...]
    uh[...] = jnp.dot(xx, uW[...], preferred_element_type=jnp.float32) + ub[...]
    vh[...] = jnp.dot(xx, vW[...], preferred_element_type=jnp.float32) + vb[...]


_DNT = (((0,), (0,)), ((), ()))


def _a_body(nmeT, W1, b1, a):
    a[...] = (lax.dot_general(nmeT[...], W1[...], _DNT,
                              preferred_element_type=jnp.float32) + b1[...])


def _sc_body(a_hbm, qh_hbm, rh_hbm, vh_hbm, src_hbm, dst_hbm,
             zeros_hbm,
             ehat_hbm, aggr_hbm,
             srcb, dstb, abuf, vbuf,
             aggr_sp, sem_i, sem_g0, sem_g1):
    sem_gs = (sem_g0, sem_g1)
    c = lax.axis_index("c")
    s = lax.axis_index("s")
    wid = s * NC + c

    @pl.when(s == 0)
    def _():
        pltpu.sync_copy(zeros_hbm, aggr_sp)

    plsc.subcore_barrier()

    def base_of(g):
        return (wid + NW * g) * CH

    def cond(g):
        return wid + NW * g < NCH

    def fire_ia(g, b):
        base = base_of(g)
        pltpu.async_copy(src_hbm.at[pl.ds(base, CH)], srcb.at[b], sem_i)
        pltpu.async_copy(dst_hbm.at[pl.ds(base, CH)], dstb.at[b], sem_i)
        pltpu.async_copy(a_hbm.at[pl.ds(base, CH)], abuf.at[b], sem_i)

    def wait_ia(b):
        pltpu.make_async_copy(src_hbm.at[pl.ds(0, CH)], srcb.at[b],
                              sem_i).wait()
        pltpu.make_async_copy(dst_hbm.at[pl.ds(0, CH)], dstb.at[b],
                              sem_i).wait()
        pltpu.make_async_copy(a_hbm.at[pl.ds(0, CH)], abuf.at[b],
                              sem_i).wait()

    def fire_gathers(b):
        pltpu.async_copy(qh_hbm.at[srcb.at[b]], abuf.at[b], sem_gs[b],
                         add=True)
        pltpu.async_copy(rh_hbm.at[dstb.at[b]], abuf.at[b], sem_gs[b],
                         add=True)
        pltpu.async_copy(vh_hbm.at[dstb.at[b]], vbuf.at[b], sem_gs[b])

    def wait_gathers(b):
        for _ in range(3):
            pltpu.make_async_copy(qh_hbm.at[pl.ds(0, CH)], vbuf.at[b],
                                  sem_gs[b]).wait()

    fire_ia(0, 0)
    wait_ia(0)
    fire_gathers(0)
    fire_ia(1, 1)

    def pair_body(i, _):
        for b in range(2):
            g = 2 * i + b

            @pl.when(cond(g + 1))
            def _():
                wait_ia(1 - b)
                fire_gathers(1 - b)

            @pl.when(cond(g))
            def _():
                wait_gathers(b)

                def edge_body(e, _):
                    for j in range(H // 16):
                        sl = pl.ds(16 * j, 16)
                        vbuf[b, e, sl] = vbuf[b, e, sl] / (
                            1.0 + jnp.exp(-abuf[b, e, sl]))
                    return 0

                lax.fori_loop(0, CH, edge_body, 0)
                base = base_of(g)
                pltpu.sync_copy(abuf.at[b], ehat_hbm.at[pl.ds(base, CH)])
                pltpu.sync_copy(vbuf.at[b], aggr_sp.at[srcb.at[b]], add=True)

            @pl.when(cond(g + 2))
            def _():
                fire_ia(g + 2, b)
        return 0

    lax.fori_loop(0, (NCH // NW + 2) // 2, pair_body, 0)

    plsc.subcore_barrier()

    @pl.when(s == 0)
    def _():
        pltpu.sync_copy(aggr_sp, aggr_hbm.at[c])


def _stats_body(ehat, src, starts, ssum, ssq, cnt):
    i = pl.program_id(0)

    @pl.when(i == 0)
    def _():
        ssum[...] = jnp.zeros((B, H), jnp.float32)
        ssq[...] = jnp.zeros((B, H), jnp.float32)
        cnt[...] = jnp.zeros((B, H), jnp.float32)

    sv = src[...]
    sv2 = sv[:, None]
    eb = jnp.where(sv2 >= 0, ehat[...], 0.0)
    seg = _bseg_of(sv, starts)
    iota = lax.broadcasted_iota(jnp.int32, (BE, B), 1)
    oh = ((seg[:, None] == iota) & (sv2 >= 0)).astype(jnp.float32)
    dn = (((0,), (0,)), ((), ()))
    ssum[...] += lax.dot_general(oh, eb, dn, preferred_element_type=jnp.float32)
    ssq[...] += lax.dot_general(oh, eb * eb, dn,
                                preferred_element_type=jnp.float32)
    cnt[...] += jnp.broadcast_to(jnp.sum(oh, axis=0)[:, None], (B, H))


def _h_body(feat, uh, p0, p1, batch, gw, gb, gms, hout):
    x = uh[...] + p0[...] + p1[...]
    seg = batch[...][:N]
    iota = lax.broadcasted_iota(jnp.int32, (N, B), 1)
    oh = (seg[:, None] == iota).astype(jnp.float32)
    dn = (((0,), (0,)), ((), ()))
    cntc = jnp.maximum(jnp.sum(oh, axis=0), 1.0)[:, None]
    mean = lax.dot_general(oh, x, dn, preferred_element_type=jnp.float32) / cntc
    out = x - jnp.dot(oh, mean, preferred_element_type=jnp.float32) * gms[...]
    var = lax.dot_general(oh, out * out, dn,
                          preferred_element_type=jnp.float32) / cntc
    rstd = 1.0 / jnp.sqrt(var + 1e-5)
    gn = out * jnp.dot(oh, rstd, preferred_element_type=jnp.float32) * gw[...] \
        + gb[...]
    hout[...] = feat[...] + jax.nn.relu(gn)


def _f_body(ehat, src, starts, nmeT, ssum, ssq, cnt, te, gw, gb, gms, lnw, lnb,
            oW, ob, etW, etb, eout):
    cntc = jnp.maximum(cnt[...], 1.0)
    m = ssum[...] / cntc
    ms = gms[...]
    var = ssq[...] / cntc - 2.0 * ms * m * m + ms * ms * m * m
    rstd = 1.0 / jnp.sqrt(var + 1e-5)
    ascale = rstd * gw[...]
    coff = gb[...] - m * ms * ascale

    seg = _bseg_of(src[...], starts)
    iota = lax.broadcasted_iota(jnp.int32, (BE, B), 1)
    oh = (seg[:, None] == iota).astype(jnp.float32)
    x = ehat[...] * jnp.dot(oh, ascale, preferred_element_type=jnp.float32) \
        + jnp.dot(oh, coff, preferred_element_type=jnp.float32)
    x = jax.nn.relu(x)
    x = x + jnp.dot(oh, te[...], preferred_element_type=jnp.float32)

    mu = jnp.mean(x, axis=-1, keepdims=True)
    vv = jnp.mean((x - mu) ** 2, axis=-1, keepdims=True)
    ln = (x - mu) / jnp.sqrt(vv + 1e-5) * lnw[...] + lnb[...]
    sl = ln * jax.nn.sigmoid(ln)
    y = jnp.dot(sl.astype(jnp.bfloat16), oW[...].astype(jnp.bfloat16),
                preferred_element_type=jnp.float32) + ob[...]
    noise = lax.dot_general(nmeT[...], etW[...], _DNT,
                            preferred_element_type=jnp.float32) + etb[...]
    eout[...] = noise + y


def _sc_stage(a, qh, rh, vh, src, dst):
    f32 = jnp.float32
    mesh = plsc.VectorSubcoreMesh(core_axis_name="c", subcore_axis_name="s",
                                  num_cores=NC, num_subcores=NS)
    sc = functools.partial(
        pl.kernel, mesh=mesh,
        out_type=[jax.ShapeDtypeStruct((E2, H), f32),
                  jax.ShapeDtypeStruct((NC, N, H), f32)],
        scratch_types=[
            pltpu.VMEM((2, CH), jnp.int32),
            pltpu.VMEM((2, CH), jnp.int32),
            pltpu.VMEM((2, CH, H), f32),
            pltpu.VMEM((2, CH, H), f32),
            pltpu.VMEM_SHARED((N, H), f32),
            pltpu.SemaphoreType.DMA,
            pltpu.SemaphoreType.DMA,
            pltpu.SemaphoreType.DMA,
        ],
    )(_sc_body)
    zeros = jnp.zeros((N, H), f32)
    return sc(a, qh, rh, vh, src, dst, zeros)


def _full(shape):
    return pl.BlockSpec(shape, lambda *_: tuple(0 for _ in shape))


def kernel(features, edge_mapping_idx, noise_mapping_emb, time_emb, batch,
           params):
    p = params
    f32 = jnp.float32
    src = edge_mapping_idx[0]
    dst = edge_mapping_idx[1]
    r1 = lambda a: a.reshape(1, -1)

    batch_pf = jnp.pad(batch, (0, NPAD - N),
                       constant_values=B + 1).astype(f32).reshape(NPAD, 1)
    W1, b1, te, starts = pl.pallas_call(
        _pp_body,
        grid=(1,),
        in_specs=[_full((NOISE, H)), _full((H, H)), _full((1, H)),
                  _full((B, H)), _full((H, H)), _full((1, H)),
                  _full((NPAD, 1))],
        out_specs=[_full((NOISE, H)), _full((1, H)), _full((B, H)),
                   _full((1, H))],
        out_shape=[jax.ShapeDtypeStruct((NOISE, H), f32),
                   jax.ShapeDtypeStruct((1, H), f32),
                   jax.ShapeDtypeStruct((B, H), f32),
                   jax.ShapeDtypeStruct((1, H), f32)],
    )(p['et_W'], p['P_W'], r1(p['et_b']), time_emb, p['t_W'], r1(p['t_b']),
      batch_pf)
    b1 = b1 + r1(p['P_b'])

    row = pl.BlockSpec((NR, H), lambda i: (i, 0))
    qh, rh, uh, vh = pl.pallas_call(
        _node_body,
        grid=(N // NR,),
        in_specs=[row] + [_full((H, H))] * 4 + [_full((1, H))] * 4,
        out_specs=[row] * 4,
        out_shape=[jax.ShapeDtypeStruct((N, H), f32)] * 4,
    )(features, p['Q_W'], p['R_W'], p['U_W'], p['V_W'],
      r1(p['Q_b']), r1(p['R_b']), r1(p['U_b']), r1(p['V_b']))

    src_p = jnp.pad(src, (0, E2 - E), constant_values=-1)
    nmeT = noise_mapping_emb.T
    ncol = pl.BlockSpec((NOISE, BE), lambda i: (0, i))
    a = pl.pallas_call(
        _a_body,
        grid=(NEB,),
        in_specs=[ncol, _full((NOISE, H)), _full((1, H))],
        out_specs=pl.BlockSpec((BE, H), lambda i: (i, 0)),
        out_shape=jax.ShapeDtypeStruct((E2, H), f32),
    )(nmeT, W1, b1)

    ehat, aggr2 = _sc_stage(a, qh, rh, vh, src, dst)

    erow = pl.BlockSpec((BE, H), lambda i: (i, 0))
    irow = pl.BlockSpec((BE,), lambda i: (i,))
    acc = pl.BlockSpec((B, H), lambda i: (0, 0))
    ssum, ssq, cnt = pl.pallas_call(
        _stats_body,
        grid=(NEB,),
        in_specs=[erow, irow, _full((1, H))],
        out_specs=[acc] * 3,
        out_shape=[jax.ShapeDtypeStruct((B, H), f32)] * 3,
    )(ehat, src_p, starts)

    batch_p = jnp.pad(batch, (0, NPAD - N), constant_values=B + 1)
    hout = pl.pallas_call(
        _h_body,
        grid=(1,),
        in_specs=[_full((N, H))] * 4 + [_full((NPAD,))] + [_full((1, H))] * 3,
        out_specs=_full((N, H)),
        out_shape=jax.ShapeDtypeStruct((N, H), f32),
    )(features, uh, aggr2[0], aggr2[1], batch_p,
      r1(p['gn_h_w']), r1(p['gn_h_b']), r1(p['gn_h_ms']))

    eout = pl.pallas_call(
        _f_body,
        grid=(NEB,),
        in_specs=[erow, irow, _full((1, H)), ncol,
                  _full((B, H)), _full((B, H)), _full((B, H)), _full((B, H)),
                  _full((1, H)), _full((1, H)), _full((1, H)),
                  _full((1, H)), _full((1, H)),
                  _full((H, H)), _full((1, H)),
                  _full((NOISE, H)), _full((1, H))],
        out_specs=erow,
        out_shape=jax.ShapeDtypeStruct((E, H), f32),
    )(ehat, src_p, starts, nmeT, ssum, ssq, cnt, te,
      r1(p['gn_e_w']), r1(p['gn_e_b']), r1(p['gn_e_ms']),
      r1(p['ln_w']), r1(p['ln_b']), p['o_W'], r1(p['o_b']),
      p['et_W'], r1(p['et_b']))

    return (hout, eout)
